# baseline (speedup 1.0000x reference)
import functools

import jax
import jax.numpy as jnp
from jax import lax
from jax.experimental import pallas as pl
from jax.experimental.pallas import tpu as pltpu
from jax.experimental.pallas import tpu_sc as plsc

_B = 16384
_K = 5
_D = 64
_NC = 2
_NS = 16
_NW = _NC * _NS
_BPW = _B // _NW
_C = 128
_NCHUNK = _BPW // _C
_NSCORE = 1 + _K
_L = 16
_U = 2 * _D

_W = 512
_HBLK = 977
_HALF = _HBLK * _W


def _tp_body(xa_ref, xb_ref, o_ref):
    eye = (lax.broadcasted_iota(jnp.int32, (_D, _D), 0) ==
           lax.broadcasted_iota(jnp.int32, (_D, _D), 1)).astype(jnp.float32)
    dn = (((0,), (0,)), ((), ()))
    o_ref[:, 0:_D] = lax.dot_general(
        xa_ref[...], eye, dn, preferred_element_type=jnp.float32)
    o_ref[:, _D:_U] = lax.dot_general(
        xb_ref[...], eye, dn, preferred_element_type=jnp.float32)


_tc_pack = pl.pallas_call(
    _tp_body,
    grid=(_HBLK,),
    in_specs=[pl.BlockSpec((_D, _W), lambda g: (0, g)),
              pl.BlockSpec((_D, _W), lambda g: (0, g + _HBLK))],
    out_specs=pl.BlockSpec((_W, _U), lambda g: (g, 0)),
    out_shape=jax.ShapeDtypeStruct((_HALF, _U), jnp.float32),
)


@functools.partial(
    pl.kernel,
    out_type=jax.ShapeDtypeStruct((_NW, _NSCORE, _BPW), jnp.float32),
    mesh=plsc.VectorSubcoreMesh(core_axis_name="c", subcore_axis_name="s"),
    scratch_types=[
        pltpu.VMEM((1, _C), jnp.int32),
        pltpu.VMEM((1, _C), jnp.int32),
        pltpu.VMEM((1, _C), jnp.int32),
        pltpu.VMEM((1, _C), jnp.int32),
        pltpu.VMEM((_K, _C), jnp.int32),
        pltpu.VMEM((_K, _C), jnp.int32),
        pltpu.VMEM((_C, _U), jnp.float32),
        pltpu.VMEM((_C, _U), jnp.float32),
        pltpu.VMEM((_K * _C, _U), jnp.float32),
        pltpu.VMEM((_NSCORE, _BPW), jnp.float32),
        pltpu.SemaphoreType.DMA,
    ],
    compiler_params=pltpu.CompilerParams(needs_layout_passes=False),
)
def _sc_scores(cen_u_hbm, cen_p_hbm, ctx_u_hbm, ctx_p_hbm, neg_u_hbm,
               neg_p_hbm, ctr_tab, ctx_tab, out_hbm,
               cen_u, cen_p, ctx_u, ctx_p, neg_u, neg_p,
               cen_rows, ctx_rows, neg_rows, scores, sem):
    wid = lax.axis_index("s") * _NC + lax.axis_index("c")
    for c in range(_NCHUNK):
        blk = wid * _NCHUNK + c
        pltpu.sync_copy(cen_u_hbm.at[blk], cen_u)
        pltpu.sync_copy(cen_p_hbm.at[blk], cen_p)
        pltpu.sync_copy(ctx_u_hbm.at[blk], ctx_u)
        pltpu.sync_copy(ctx_p_hbm.at[blk], ctx_p)
        pltpu.sync_copy(neg_u_hbm.at[blk], neg_u)
        pltpu.sync_copy(neg_p_hbm.at[blk], neg_p)
        copies = [
            pltpu.async_copy(ctr_tab.at[cen_u.at[0]], cen_rows, sem),
            pltpu.async_copy(ctx_tab.at[ctx_u.at[0]], ctx_rows, sem),
        ]
        for j in range(_K):
            copies.append(pltpu.async_copy(
                ctx_tab.at[neg_u.at[j]], neg_rows.at[pl.ds(j * _C, _C)],
                sem))
        for cp in copies:
            cp.wait()
        for g in range(_C // _L):
            off = g * _L
            rows = lax.iota(jnp.int32, _L) + off
            pc = cen_p[0, pl.ds(off, _L)]
            px = ctx_p[0, pl.ds(off, _L)]
            nrows = [rows * _K + k for k in range(_K)]
            pn = [neg_p[k, pl.ds(off, _L)] for k in range(_K)]

            def dbody(d, accs, rows=rows, pc=pc, px=px, nrows=nrows, pn=pn):
                dv = jnp.full((_L,), d, jnp.int32)
                cen = plsc.load_gather(cen_rows, [rows, pc + dv])
                ctx = plsc.load_gather(ctx_rows, [rows, px + dv])
                new = [accs[0] + cen * ctx]
                for k in range(_K):
                    nk = plsc.load_gather(neg_rows, [nrows[k], pn[k] + dv])
                    new.append(accs[k + 1] + cen * nk)
                return tuple(new)

            accs = lax.fori_loop(
                0, _D, dbody,
                tuple(jnp.zeros((_L,), jnp.float32) for _ in range(_NSCORE)))
            for s in range(_NSCORE):
                scores[s, pl.ds(c * _C + off, _L)] = accs[s]
    pltpu.sync_copy(scores, out_hbm.at[wid])


def _softplus(x):
    return jnp.maximum(x, 0.0) + jnp.log1p(jnp.exp(-jnp.abs(x)))


def _loss_body(s_ref, o_ref):
    x = s_ref[...]
    pos = x[:, 0, :]
    neg = x[:, 1:, :]
    total = jnp.sum(_softplus(-pos)) + jnp.sum(_softplus(neg))
    o_ref[0, 0] = total * (1.0 / _B)


_loss = pl.pallas_call(
    _loss_body,
    out_shape=jax.ShapeDtypeStruct((1, 1), jnp.float32),
    out_specs=pl.BlockSpec(memory_space=pltpu.SMEM),
)


def _shape_idx(u, p, rows):
    return (u.reshape(_B // _C, rows, _C).astype(jnp.int32),
            (p * _D).reshape(_B // _C, rows, _C).astype(jnp.int32))


def kernel(pairs, negs, center_weight, context_weight):
    centers = pairs[:, 0].astype(jnp.int32)
    cen_u, cen_p = _shape_idx(centers % _HALF, centers // _HALF, 1)
    ctx = pairs[:, 1].astype(jnp.int32)
    ctx_u, ctx_p = _shape_idx(ctx >> 1, ctx & 1, 1)
    negs = negs.astype(jnp.int32)
    neg_u, neg_p = _shape_idx(negs >> 1, negs & 1, _K)
    ctr_tab = _tc_pack(center_weight.T, center_weight.T)
    ctx_tab = context_weight.reshape(-1, _U)
    scores = _sc_scores(cen_u, cen_p, ctx_u, ctx_p, neg_u, neg_p,
                        ctr_tab, ctx_tab)
    return _loss(scores)[0, 0]


# speedup vs baseline: 1.1032x; 1.1032x over previous
import functools

import jax
import jax.numpy as jnp
from jax import lax
from jax.experimental import pallas as pl
from jax.experimental.pallas import tpu as pltpu
from jax.experimental.pallas import tpu_sc as plsc

_B = 16384
_K = 5
_D = 64
_NC = 2
_NS = 16
_NW = _NC * _NS
_BPW = _B // _NW
_C = 128
_NCHUNK = _BPW // _C
_NSCORE = 1 + _K
_L = 16
_U = 2 * _D
_UNROLL = 4


@functools.partial(
    pl.kernel,
    out_type=jax.ShapeDtypeStruct((_NW, _NSCORE, _BPW), jnp.float32),
    mesh=plsc.VectorSubcoreMesh(core_axis_name="c", subcore_axis_name="s"),
    scratch_types=[
        pltpu.VMEM((1, _C), jnp.int32),
        pltpu.VMEM((1, _C), jnp.int32),
        pltpu.VMEM((1, _C), jnp.int32),
        pltpu.VMEM((1, _C), jnp.int32),
        pltpu.VMEM((_K, _C), jnp.int32),
        pltpu.VMEM((_K, _C), jnp.int32),
        pltpu.VMEM((_C, _U), jnp.float32),
        pltpu.VMEM((_C, _U), jnp.float32),
        pltpu.VMEM((_K * _C, _U), jnp.float32),
        pltpu.VMEM((_NSCORE, _BPW), jnp.float32),
        pltpu.SemaphoreType.DMA,
    ],
    compiler_params=pltpu.CompilerParams(needs_layout_passes=False),
)
def _sc_scores(cen_u_hbm, cen_p_hbm, ctx_u_hbm, ctx_p_hbm, neg_u_hbm,
               neg_p_hbm, ctr_tab, ctx_tab, out_hbm,
               cen_u, cen_p, ctx_u, ctx_p, neg_u, neg_p,
               cen_rows, ctx_rows, neg_rows, scores, sem):
    wid = lax.axis_index("s") * _NC + lax.axis_index("c")
    for c in range(_NCHUNK):
        blk = wid * _NCHUNK + c
        pltpu.sync_copy(cen_u_hbm.at[blk], cen_u)
        pltpu.sync_copy(cen_p_hbm.at[blk], cen_p)
        pltpu.sync_copy(ctx_u_hbm.at[blk], ctx_u)
        pltpu.sync_copy(ctx_p_hbm.at[blk], ctx_p)
        pltpu.sync_copy(neg_u_hbm.at[blk], neg_u)
        pltpu.sync_copy(neg_p_hbm.at[blk], neg_p)
        copies = [
            pltpu.async_copy(ctr_tab.at[cen_u.at[0]], cen_rows, sem),
            pltpu.async_copy(ctx_tab.at[ctx_u.at[0]], ctx_rows, sem),
        ]
        for j in range(_K):
            copies.append(pltpu.async_copy(
                ctx_tab.at[neg_u.at[j]], neg_rows.at[pl.ds(j * _C, _C)],
                sem))
        for cp in copies:
            cp.wait()
        for g in range(_C // _L):
            off = g * _L
            rows = lax.iota(jnp.int32, _L) + off
            pc = cen_p[0, pl.ds(off, _L)]
            px = ctx_p[0, pl.ds(off, _L)]
            nrows = [rows * _K + k for k in range(_K)]
            pn = [neg_p[k, pl.ds(off, _L)] for k in range(_K)]

            def dbody(i, accs, rows=rows, pc=pc, px=px, nrows=nrows, pn=pn):
                new = list(accs)
                for u in range(_UNROLL):
                    dv = jnp.full((_L,), i * _UNROLL + u, jnp.int32)
                    cen = plsc.load_gather(cen_rows, [rows, pc + dv])
                    ctx = plsc.load_gather(ctx_rows, [rows, px + dv])
                    new[0] = new[0] + cen * ctx
                    for k in range(_K):
                        nk = plsc.load_gather(
                            neg_rows, [nrows[k], pn[k] + dv])
                        new[k + 1] = new[k + 1] + cen * nk
                return tuple(new)

            accs = lax.fori_loop(
                0, _D // _UNROLL, dbody,
                tuple(jnp.zeros((_L,), jnp.float32) for _ in range(_NSCORE)))
            for s in range(_NSCORE):
                scores[s, pl.ds(c * _C + off, _L)] = accs[s]
    pltpu.sync_copy(scores, out_hbm.at[wid])


def _softplus(x):
    return jnp.maximum(x, 0.0) + jnp.log1p(jnp.exp(-jnp.abs(x)))


def _loss_body(s_ref, o_ref):
    x = s_ref[...]
    pos = x[:, 0, :]
    neg = x[:, 1:, :]
    total = jnp.sum(_softplus(-pos)) + jnp.sum(_softplus(neg))
    o_ref[0, 0] = total * (1.0 / _B)


_loss = pl.pallas_call(
    _loss_body,
    out_shape=jax.ShapeDtypeStruct((1, 1), jnp.float32),
    out_specs=pl.BlockSpec(memory_space=pltpu.SMEM),
)


def _split_idx(idx, rows):
    idx = idx.astype(jnp.int32)
    return (jnp.right_shift(idx, 1).reshape(_B // _C, rows, _C),
            (jnp.bitwise_and(idx, 1) * _D).reshape(_B // _C, rows, _C))


def kernel(pairs, negs, center_weight, context_weight):
    cen_u, cen_p = _split_idx(pairs[:, 0], 1)
    ctx_u, ctx_p = _split_idx(pairs[:, 1], 1)
    neg_u, neg_p = _split_idx(negs, _K)
    ctr_tab = center_weight.reshape(-1, _U)
    ctx_tab = context_weight.reshape(-1, _U)
    scores = _sc_scores(cen_u, cen_p, ctx_u, ctx_p, neg_u, neg_p,
                        ctr_tab, ctx_tab)
    return _loss(scores)[0, 0]


# speedup vs baseline: 1.1061x; 1.0026x over previous
import functools

import jax
import jax.numpy as jnp
from jax import lax
from jax.experimental import pallas as pl
from jax.experimental.pallas import tpu as pltpu
from jax.experimental.pallas import tpu_sc as plsc

_B = 16384
_K = 5
_D = 64
_NC = 2
_NS = 16
_NW = _NC * _NS
_BPW = _B // _NW
_C = 64
_NCHUNK = _BPW // _C
_NSCORE = 1 + _K
_L = 16
_U = 2 * _D
_UNROLL = 4


@functools.partial(
    pl.kernel,
    out_type=jax.ShapeDtypeStruct((_NW, _NSCORE, _BPW), jnp.float32),
    mesh=plsc.VectorSubcoreMesh(core_axis_name="c", subcore_axis_name="s"),
    scratch_types=[
        [pltpu.VMEM((1, _C), jnp.int32)] * 2,
        [pltpu.VMEM((1, _C), jnp.int32)] * 2,
        [pltpu.VMEM((1, _C), jnp.int32)] * 2,
        [pltpu.VMEM((1, _C), jnp.int32)] * 2,
        [pltpu.VMEM((_K, _C), jnp.int32)] * 2,
        [pltpu.VMEM((_K, _C), jnp.int32)] * 2,
        [pltpu.VMEM((_C, _U), jnp.float32)] * 2,
        [pltpu.VMEM((_C, _U), jnp.float32)] * 2,
        [pltpu.VMEM((_K * _C, _U), jnp.float32)] * 2,
        pltpu.VMEM((_NSCORE, _BPW), jnp.float32),
        [pltpu.SemaphoreType.DMA] * 2,
    ],
    compiler_params=pltpu.CompilerParams(needs_layout_passes=False),
)
def _sc_scores(cen_u_hbm, cen_p_hbm, ctx_u_hbm, ctx_p_hbm, neg_u_hbm,
               neg_p_hbm, ctr_tab, ctx_tab, out_hbm,
               cen_u, cen_p, ctx_u, ctx_p, neg_u, neg_p,
               cen_rows, ctx_rows, neg_rows, scores, sem):
    wid = lax.axis_index("s") * _NC + lax.axis_index("c")

    def fire(c):
        p = c % 2
        blk = wid * _NCHUNK + c
        pltpu.sync_copy(cen_u_hbm.at[blk], cen_u[p])
        pltpu.sync_copy(cen_p_hbm.at[blk], cen_p[p])
        pltpu.sync_copy(ctx_u_hbm.at[blk], ctx_u[p])
        pltpu.sync_copy(ctx_p_hbm.at[blk], ctx_p[p])
        pltpu.sync_copy(neg_u_hbm.at[blk], neg_u[p])
        pltpu.sync_copy(neg_p_hbm.at[blk], neg_p[p])
        copies = [
            pltpu.async_copy(ctr_tab.at[cen_u[p].at[0]], cen_rows[p],
                             sem[p]),
            pltpu.async_copy(ctx_tab.at[ctx_u[p].at[0]], ctx_rows[p],
                             sem[p]),
        ]
        for j in range(_K):
            copies.append(pltpu.async_copy(
                ctx_tab.at[neg_u[p].at[j]],
                neg_rows[p].at[pl.ds(j * _C, _C)], sem[p]))
        return copies

    pend = fire(0)
    for c in range(_NCHUNK):
        p = c % 2
        for cp in pend:
            cp.wait()
        if c + 1 < _NCHUNK:
            pend = fire(c + 1)
        for g in range(_C // _L):
            off = g * _L
            rows = lax.iota(jnp.int32, _L) + off
            pc = cen_p[p][0, pl.ds(off, _L)]
            px = ctx_p[p][0, pl.ds(off, _L)]
            nrows = [rows * _K + k for k in range(_K)]
            pn = [neg_p[p][k, pl.ds(off, _L)] for k in range(_K)]

            def dbody(i, accs, p=p, rows=rows, pc=pc, px=px, nrows=nrows,
                      pn=pn):
                new = list(accs)
                for u in range(_UNROLL):
                    dv = jnp.full((_L,), i * _UNROLL + u, jnp.int32)
                    cen = plsc.load_gather(cen_rows[p], [rows, pc + dv])
                    ctx = plsc.load_gather(ctx_rows[p], [rows, px + dv])
                    new[0] = new[0] + cen * ctx
                    for k in range(_K):
                        nk = plsc.load_gather(
                            neg_rows[p], [nrows[k], pn[k] + dv])
                        new[k + 1] = new[k + 1] + cen * nk
                return tuple(new)

            accs = lax.fori_loop(
                0, _D // _UNROLL, dbody,
                tuple(jnp.zeros((_L,), jnp.float32) for _ in range(_NSCORE)))
            for s in range(_NSCORE):
                scores[s, pl.ds(c * _C + off, _L)] = accs[s]
    pltpu.sync_copy(scores, out_hbm.at[wid])


def _softplus(x):
    return jnp.maximum(x, 0.0) + jnp.log1p(jnp.exp(-jnp.abs(x)))


def _loss_body(s_ref, o_ref):
    x = s_ref[...]
    pos = x[:, 0, :]
    neg = x[:, 1:, :]
    total = jnp.sum(_softplus(-pos)) + jnp.sum(_softplus(neg))
    o_ref[0, 0] = total * (1.0 / _B)


_loss = pl.pallas_call(
    _loss_body,
    out_shape=jax.ShapeDtypeStruct((1, 1), jnp.float32),
    out_specs=pl.BlockSpec(memory_space=pltpu.SMEM),
)


def _split_idx(idx, rows):
    idx = idx.astype(jnp.int32)
    return (jnp.right_shift(idx, 1).reshape(_B // _C, rows, _C),
            (jnp.bitwise_and(idx, 1) * _D).reshape(_B // _C, rows, _C))


def kernel(pairs, negs, center_weight, context_weight):
    cen_u, cen_p = _split_idx(pairs[:, 0], 1)
    ctx_u, ctx_p = _split_idx(pairs[:, 1], 1)
    neg_u, neg_p = _split_idx(negs, _K)
    ctr_tab = center_weight.reshape(-1, _U)
    ctx_tab = context_weight.reshape(-1, _U)
    scores = _sc_scores(cen_u, cen_p, ctx_u, ctx_p, neg_u, neg_p,
                        ctr_tab, ctx_tab)
    return _loss(scores)[0, 0]
